# baseline (speedup 1.0000x reference)
import functools

import jax
import jax.numpy as jnp
from jax import lax
from jax.experimental import pallas as pl
from jax.experimental.pallas import tpu as pltpu
from jax.experimental.pallas import tpu_sc as plsc

C, HF, WF = 256, 64, 64
NB = 5000
OH, OW = 7, 7
IMG_H, IMG_W = 1024.0, 1024.0

NPOS = OH * OW
CSTRIDE = 52
NROWS = 4 * CSTRIDE
ROW_WORDS = C * NPOS
NW = 32
BPW = 160
NB_PAD = NW * BPW
CHUNK_STARTS = (0, 16, 32, 33)

SX = (WF - 1) / (IMG_W - 1)
SY = (HF - 1) / (IMG_H - 1)


def _splat_f32(x):
  return jnp.full((16,), x, dtype=jnp.float32)


def _splat_i32(x):
  return jnp.full((16,), x, dtype=jnp.int32)


def _coords(boxes_v, k, tpos_v, st):
  bv = boxes_v[k]
  xc = _splat_f32(bv[0])
  yc = _splat_f32(bv[1])
  w = _splat_f32(bv[2])
  h = _splat_f32(bv[3])
  x0 = xc - w * 0.5
  y0 = yc - h * 0.5
  typ = tpos_v[pl.ds(st, 16)]
  txp = tpos_v[pl.ds(64 + st, 16)]
  ys = jnp.minimum(jnp.maximum((y0 + h * typ) * SY, 0.0), HF - 1.0)
  xs = jnp.minimum(jnp.maximum((x0 + w * txp) * SX, 0.0), WF - 1.0)
  y0i = ys.astype(jnp.int32)
  x0i = xs.astype(jnp.int32)
  wy = ys - y0i.astype(jnp.float32)
  wx = xs - x0i.astype(jnp.float32)
  y1i = jnp.minimum(y0i + 1, HF - 1)
  x1i = jnp.minimum(x0i + 1, WF - 1)
  idx = (y0i * WF + x0i, y0i * WF + x1i, y1i * WF + x0i, y1i * WF + x1i)
  wts = ((1.0 - wy) * (1.0 - wx), (1.0 - wy) * wx, wy * (1.0 - wx), wy * wx)
  return idx, wts


def _sc_body(table_h, boxes_h, tpos_h, out_h,
             boxes_v, tpos_v, idx_v, g_v, out_v, sem0):
  cid = lax.axis_index("c")
  sid = lax.axis_index("s")
  wid = sid * 2 + cid
  start = wid * BPW
  count = jnp.minimum(BPW, jnp.maximum(NB - start, 0))

  pltpu.sync_copy(tpos_h, tpos_v)
  pltpu.sync_copy(boxes_h.at[pl.ds(start, BPW)], boxes_v)

  zeros16 = _splat_i32(0)
  for i in range(NROWS // 16):
    idx_v[pl.ds(16 * i, 16)] = zeros16

  iota16 = lax.iota(jnp.int32, 16)

  def blend(k):
    wts_l = []
    krow_l = []
    for st in CHUNK_STARTS:
      _, wts = _coords(boxes_v, k, tpos_v, st)
      p_vec = iota16 + st
      wts_l.append(wts)
      krow_l.append(tuple(p_vec + corner * CSTRIDE for corner in range(4)))

    @plsc.parallel_loop(0, C, 1, unroll=2)
    def blend_c(c):
      cvec = _splat_i32(c)
      for wts, krow, st in zip(wts_l, krow_l, CHUNK_STARTS):
        acc = wts[0] * plsc.load_gather(g_v, [krow[0], cvec])
        acc += wts[1] * plsc.load_gather(g_v, [krow[1], cvec])
        acc += wts[2] * plsc.load_gather(g_v, [krow[2], cvec])
        acc += wts[3] * plsc.load_gather(g_v, [krow[3], cvec])
        out_v[pl.ds(c * NPOS + st, 16)] = acc

  def box_body(k, carry):
    for st in CHUNK_STARTS:
      idx, _ = _coords(boxes_v, k, tpos_v, st)
      for corner in range(4):
        idx_v[pl.ds(corner * CSTRIDE + st, 16)] = idx[corner]
    for j in range(13):
      pltpu.async_copy(table_h.at[idx_v.at[pl.ds(16 * j, 16)]],
                       g_v.at[pl.ds(16 * j, 16)], sem0)
    for j in range(13):
      pltpu.make_async_copy(table_h.at[idx_v.at[pl.ds(16 * j, 16)]],
                            g_v.at[pl.ds(16 * j, 16)], sem0).wait()
    blend(k)
    pltpu.sync_copy(out_v, out_h.at[start + k])
    return carry

  lax.fori_loop(0, count, box_body, 0)


_mesh = plsc.VectorSubcoreMesh(core_axis_name="c", subcore_axis_name="s")

_sc_call = functools.partial(
    pl.kernel, mesh=_mesh,
    out_type=jax.ShapeDtypeStruct((NB, ROW_WORDS), jnp.float32),
    scratch_types=[
        pltpu.VMEM((BPW, 16), jnp.float32),
        pltpu.VMEM((128,), jnp.float32),
        pltpu.VMEM((NROWS,), jnp.int32),
        pltpu.VMEM((NROWS, C), jnp.float32),
        pltpu.VMEM((ROW_WORDS,), jnp.float32),
        pltpu.SemaphoreType.DMA,
    ],
    compiler_params=pltpu.CompilerParams(needs_layout_passes=False),
)(_sc_body)


def _tpos_host():
  ty = jnp.linspace(0.0, 1.0, OH)
  tx = jnp.linspace(0.0, 1.0, OW)
  p = jnp.arange(64)
  typ = jnp.where(p < NPOS, ty[jnp.minimum(p // OW, OH - 1)], 0.0)
  txp = jnp.where(p < NPOS, tx[p % OW], 0.0)
  return jnp.concatenate([typ, txp]).astype(jnp.float32)


def kernel(cnn_features, boxes):
  feat = cnn_features[0]
  table = jnp.transpose(feat, (1, 2, 0)).reshape(HF * WF, C)
  boxes_p = jnp.zeros((NB_PAD, 16), jnp.float32).at[:NB, :4].set(boxes)
  out = _sc_call(table, boxes_p, _tpos_host())
  return out.reshape(NB, C, OH, OW)


# speedup vs baseline: 1.7322x; 1.7322x over previous
import functools

import jax
import jax.numpy as jnp
from jax import lax
from jax.experimental import pallas as pl
from jax.experimental.pallas import tpu as pltpu
from jax.experimental.pallas import tpu_sc as plsc

C, HF, WF = 256, 64, 64
NB = 5000
OH, OW = 7, 7
IMG_H, IMG_W = 1024.0, 1024.0

NPOS = OH * OW
CSTRIDE = 52
NROWS = 4 * CSTRIDE
ROW_WORDS = C * NPOS
NW = 32
BPW = 160
NB_PAD = NW * BPW
CHUNK_STARTS = (0, 16, 32, 33)

SX = (WF - 1) / (IMG_W - 1)
SY = (HF - 1) / (IMG_H - 1)


def _splat_f32(x):
  return jnp.full((16,), x, dtype=jnp.float32)


def _splat_i32(x):
  return jnp.full((16,), x, dtype=jnp.int32)


def _coords(boxes_v, k, tpos_v, st):
  bv = boxes_v[k]
  xc = _splat_f32(bv[0])
  yc = _splat_f32(bv[1])
  w = _splat_f32(bv[2])
  h = _splat_f32(bv[3])
  x0 = xc - w * 0.5
  y0 = yc - h * 0.5
  typ = tpos_v[pl.ds(st, 16)]
  txp = tpos_v[pl.ds(64 + st, 16)]
  ys = jnp.minimum(jnp.maximum((y0 + h * typ) * SY, 0.0), HF - 1.0)
  xs = jnp.minimum(jnp.maximum((x0 + w * txp) * SX, 0.0), WF - 1.0)
  y0i = ys.astype(jnp.int32)
  x0i = xs.astype(jnp.int32)
  wy = ys - y0i.astype(jnp.float32)
  wx = xs - x0i.astype(jnp.float32)
  y1i = jnp.minimum(y0i + 1, HF - 1)
  x1i = jnp.minimum(x0i + 1, WF - 1)
  idx = (y0i * WF + x0i, y0i * WF + x1i, y1i * WF + x0i, y1i * WF + x1i)
  wts = ((1.0 - wy) * (1.0 - wx), (1.0 - wy) * wx, wy * (1.0 - wx), wy * wx)
  return idx, wts


def _sc_body(table_h, boxes_h, tpos_h, out_h,
             boxes_v, tpos_v, idx_v, g_v, out_v, wbuf_v, sem0):
  cid = lax.axis_index("c")
  sid = lax.axis_index("s")
  wid = sid * 2 + cid
  start = wid * BPW
  count = jnp.minimum(BPW, jnp.maximum(NB - start, 0))

  pltpu.sync_copy(tpos_h, tpos_v)
  pltpu.sync_copy(boxes_h.at[pl.ds(start, BPW)], boxes_v)

  zeros16 = _splat_i32(0)
  for i in range(NROWS // 16):
    idx_v[pl.ds(16 * i, 16)] = zeros16

  iota16 = lax.iota(jnp.int32, 16)

  iota49 = iota16 * NPOS

  def blend(k):
    @plsc.parallel_loop(0, NPOS, 1, unroll=2)
    def p_body(p):
      w = []
      for corner in range(4):
        wv = wbuf_v[corner, pl.ds(p, 16)]
        w.append(_splat_f32(wv[0]))
      for cc in range(16):
        sl = pl.ds(cc * 16, 16)
        acc = w[0] * g_v[p, sl]
        acc += w[1] * g_v[p + CSTRIDE, sl]
        acc += w[2] * g_v[p + 2 * CSTRIDE, sl]
        acc += w[3] * g_v[p + 3 * CSTRIDE, sl]
        plsc.store_scatter(out_v, [iota49 + (cc * 16 * NPOS + p)], acc)

  def box_body(k, carry):
    for st in CHUNK_STARTS:
      idx, wts = _coords(boxes_v, k, tpos_v, st)
      for corner in range(4):
        idx_v[pl.ds(corner * CSTRIDE + st, 16)] = idx[corner]
        wbuf_v[corner, pl.ds(st, 16)] = wts[corner]
    for j in range(13):
      pltpu.async_copy(table_h.at[idx_v.at[pl.ds(16 * j, 16)]],
                       g_v.at[pl.ds(16 * j, 16)], sem0)
    for j in range(13):
      pltpu.make_async_copy(table_h.at[idx_v.at[pl.ds(16 * j, 16)]],
                            g_v.at[pl.ds(16 * j, 16)], sem0).wait()
    blend(k)
    pltpu.sync_copy(out_v, out_h.at[start + k])
    return carry

  lax.fori_loop(0, count, box_body, 0)


_mesh = plsc.VectorSubcoreMesh(core_axis_name="c", subcore_axis_name="s")

_sc_call = functools.partial(
    pl.kernel, mesh=_mesh,
    out_type=jax.ShapeDtypeStruct((NB, ROW_WORDS), jnp.float32),
    scratch_types=[
        pltpu.VMEM((BPW, 16), jnp.float32),
        pltpu.VMEM((128,), jnp.float32),
        pltpu.VMEM((NROWS,), jnp.int32),
        pltpu.VMEM((NROWS, C), jnp.float32),
        pltpu.VMEM((ROW_WORDS,), jnp.float32),
        pltpu.VMEM((4, 64), jnp.float32),
        pltpu.SemaphoreType.DMA,
    ],
    compiler_params=pltpu.CompilerParams(needs_layout_passes=False),
)(_sc_body)


def _tpos_host():
  ty = jnp.linspace(0.0, 1.0, OH)
  tx = jnp.linspace(0.0, 1.0, OW)
  p = jnp.arange(64)
  typ = jnp.where(p < NPOS, ty[jnp.minimum(p // OW, OH - 1)], 0.0)
  txp = jnp.where(p < NPOS, tx[p % OW], 0.0)
  return jnp.concatenate([typ, txp]).astype(jnp.float32)


def kernel(cnn_features, boxes):
  feat = cnn_features[0]
  table = jnp.transpose(feat, (1, 2, 0)).reshape(HF * WF, C)
  boxes_p = jnp.zeros((NB_PAD, 16), jnp.float32).at[:NB, :4].set(boxes)
  out = _sc_call(table, boxes_p, _tpos_host())
  return out.reshape(NB, C, OH, OW)
